# baseline (speedup 1.0000x reference)
import functools

import jax
import jax.numpy as jnp
from jax import lax
from jax.experimental import pallas as pl
from jax.experimental.pallas import tpu as pltpu
from jax.experimental.pallas import tpu_sc as plsc

B = 4096
L = 200
EMB = 64
HID = 128
NCLS = 2

NC = 2
NS = 16
NW = NC * NS
RPW = B // NW
K1 = 104
K2 = L - K1


def _sc_pool_body(idx_hbm, table_hbm, out_hbm, idx_v, rows_v, out_v, sem0, sem1):
    wid = lax.axis_index("c") * NS + lax.axis_index("s")
    base = wid * RPW
    pltpu.sync_copy(idx_hbm.at[wid], idx_v)

    def fire(r, buf, sem):
        pltpu.make_async_copy(
            table_hbm.at[idx_v.at[r, pl.ds(0, K1)]],
            rows_v.at[buf, pl.ds(0, K1)], sem).start()
        pltpu.make_async_copy(
            table_hbm.at[idx_v.at[r, pl.ds(K1, K2)]],
            rows_v.at[buf, pl.ds(K1, K2)], sem).start()

    def drain(r, buf, sem):
        pltpu.make_async_copy(
            table_hbm.at[idx_v.at[r, pl.ds(0, K1)]],
            rows_v.at[buf, pl.ds(0, K1)], sem).wait()
        pltpu.make_async_copy(
            table_hbm.at[idx_v.at[r, pl.ds(K1, K2)]],
            rows_v.at[buf, pl.ds(K1, K2)], sem).wait()

    def accum(r, buf):
        def body(l, acc):
            a0, a1, a2, a3 = acc
            a0 = a0 + rows_v[buf, l, pl.ds(0, 16)]
            a1 = a1 + rows_v[buf, l, pl.ds(16, 16)]
            a2 = a2 + rows_v[buf, l, pl.ds(32, 16)]
            a3 = a3 + rows_v[buf, l, pl.ds(48, 16)]
            return (a0, a1, a2, a3)

        z = jnp.zeros((16,), jnp.float32)
        a0, a1, a2, a3 = lax.fori_loop(0, L, body, (z, z, z, z))
        out_v[r, pl.ds(0, 16)] = a0
        out_v[r, pl.ds(16, 16)] = a1
        out_v[r, pl.ds(32, 16)] = a2
        out_v[r, pl.ds(48, 16)] = a3

    fire(0, 0, sem0)

    def outer(i, _):
        r0 = 2 * i
        r1 = r0 + 1
        fire(r1, 1, sem1)
        drain(r0, 0, sem0)
        accum(r0, 0)

        @pl.when(i < RPW // 2 - 1)
        def _():
            fire(r1 + 1, 0, sem0)

        drain(r1, 1, sem1)
        accum(r1, 1)
        return 0

    lax.fori_loop(0, RPW // 2, outer, 0)
    pltpu.sync_copy(out_v, out_hbm.at[pl.ds(base, RPW)])


_sc_pool = functools.partial(
    pl.kernel,
    out_type=jax.ShapeDtypeStruct((B, EMB), jnp.float32),
    mesh=plsc.VectorSubcoreMesh(core_axis_name="c", subcore_axis_name="s"),
    scratch_types=[
        pltpu.VMEM((RPW, L), jnp.int32),
        pltpu.VMEM((2, L, EMB), jnp.float32),
        pltpu.VMEM((RPW, EMB), jnp.float32),
        pltpu.SemaphoreType.DMA,
        pltpu.SemaphoreType.DMA,
    ],
    compiler_params=pltpu.CompilerParams(use_tc_tiling_on_sc=False),
)(_sc_pool_body)


def _mlp_body(x_ref, len_ref, w1_ref, b1_ref, w2_ref, b2_ref, o_ref):
    x = x_ref[...] / len_ref[...]
    h = jnp.dot(x, w1_ref[...], preferred_element_type=jnp.float32) + b1_ref[...]
    h = jnp.maximum(h, 0.0)
    o_ref[...] = jnp.dot(h, w2_ref[...], preferred_element_type=jnp.float32) + b2_ref[...]


_mlp = pl.pallas_call(
    _mlp_body,
    out_shape=jax.ShapeDtypeStruct((B, HID), jnp.float32),
)


def kernel(batch_inputs, batch_lengths, table, W1, b1, W2, b2):
    idx = batch_inputs.reshape(NW, RPW, L)
    pooled = _sc_pool(idx, table)
    w2p = jnp.pad(W2, ((0, 0), (0, HID - NCLS)))
    b2p = jnp.pad(b2, (0, HID - NCLS)).reshape(1, HID)
    out = _mlp(pooled, batch_lengths.reshape(B, 1), W1, b1.reshape(1, HID), w2p, b2p)
    return out[:, :NCLS]


# speedup vs baseline: 1.0012x; 1.0012x over previous
import functools

import jax
import jax.numpy as jnp
from jax import lax
from jax.experimental import pallas as pl
from jax.experimental.pallas import tpu as pltpu
from jax.experimental.pallas import tpu_sc as plsc

B = 4096
L = 200
EMB = 64
HID = 128
NCLS = 2

NC = 2
NS = 16
NW = NC * NS
RPW = B // NW
K1 = 104
K2 = L - K1


def _sc_pool_body(idx_hbm, table_hbm, out_hbm, idx_v, rows_v, out_v, sem0, sem1):
    wid = lax.axis_index("c") * NS + lax.axis_index("s")
    base = wid * RPW
    pltpu.sync_copy(idx_hbm.at[pl.ds(base, RPW)], idx_v)

    def fire(r, buf, sem):
        pltpu.make_async_copy(
            table_hbm.at[idx_v.at[r, pl.ds(0, K1)]],
            rows_v.at[buf, pl.ds(0, K1)], sem).start()
        pltpu.make_async_copy(
            table_hbm.at[idx_v.at[r, pl.ds(K1, K2)]],
            rows_v.at[buf, pl.ds(K1, K2)], sem).start()

    def drain(r, buf, sem):
        pltpu.make_async_copy(
            table_hbm.at[idx_v.at[r, pl.ds(0, K1)]],
            rows_v.at[buf, pl.ds(0, K1)], sem).wait()
        pltpu.make_async_copy(
            table_hbm.at[idx_v.at[r, pl.ds(K1, K2)]],
            rows_v.at[buf, pl.ds(K1, K2)], sem).wait()

    def accum(r, buf):
        def body(l, acc):
            a0, a1, a2, a3 = acc
            a0 = a0 + rows_v[buf, l, pl.ds(0, 16)]
            a1 = a1 + rows_v[buf, l, pl.ds(16, 16)]
            a2 = a2 + rows_v[buf, l, pl.ds(32, 16)]
            a3 = a3 + rows_v[buf, l, pl.ds(48, 16)]
            return (a0, a1, a2, a3)

        z = jnp.zeros((16,), jnp.float32)
        a0, a1, a2, a3 = lax.fori_loop(0, L, body, (z, z, z, z))
        out_v[r, pl.ds(0, 16)] = a0
        out_v[r, pl.ds(16, 16)] = a1
        out_v[r, pl.ds(32, 16)] = a2
        out_v[r, pl.ds(48, 16)] = a3

    fire(0, 0, sem0)

    def outer(i, _):
        r0 = 2 * i
        r1 = r0 + 1
        fire(r1, 1, sem1)
        drain(r0, 0, sem0)
        accum(r0, 0)

        @pl.when(i < RPW // 2 - 1)
        def _():
            fire(r1 + 1, 0, sem0)

        drain(r1, 1, sem1)
        accum(r1, 1)
        return 0

    lax.fori_loop(0, RPW // 2, outer, 0)
    pltpu.sync_copy(out_v, out_hbm.at[pl.ds(base, RPW)])


_sc_pool = functools.partial(
    pl.kernel,
    out_type=jax.ShapeDtypeStruct((B, EMB), jnp.float32),
    mesh=plsc.VectorSubcoreMesh(core_axis_name="c", subcore_axis_name="s"),
    scratch_types=[
        pltpu.VMEM((RPW, L), jnp.int32),
        pltpu.VMEM((2, L, EMB), jnp.float32),
        pltpu.VMEM((RPW, EMB), jnp.float32),
        pltpu.SemaphoreType.DMA,
        pltpu.SemaphoreType.DMA,
    ],
    compiler_params=pltpu.CompilerParams(use_tc_tiling_on_sc=False),
)(_sc_pool_body)


def _mlp_body(x_ref, len_ref, w1_ref, b1_ref, w2_ref, b2_ref, o_ref):
    x = x_ref[...] / len_ref[...]
    h = jnp.dot(x, w1_ref[...], preferred_element_type=jnp.float32) + b1_ref[...]
    h = jnp.maximum(h, 0.0)
    o_ref[...] = jnp.dot(h, w2_ref[...], preferred_element_type=jnp.float32) + b2_ref[...]


_mlp = pl.pallas_call(
    _mlp_body,
    out_shape=jax.ShapeDtypeStruct((B, HID), jnp.float32),
)


def kernel(batch_inputs, batch_lengths, table, W1, b1, W2, b2):
    pooled = _sc_pool(batch_inputs, table)
    w2p = jnp.pad(W2, ((0, 0), (0, HID - NCLS)))
    b2p = jnp.pad(b2, (0, HID - NCLS)).reshape(1, HID)
    out = _mlp(pooled, batch_lengths.reshape(B, 1), W1, b1.reshape(1, HID), w2p, b2p)
    return out[:, :NCLS]
